# baseline (speedup 1.0000x reference)
import dataclasses
import functools

import jax
import jax.numpy as jnp
from jax import lax
from jax.experimental import pallas as pl
from jax.experimental.pallas import tpu as pltpu
from jax.experimental.pallas import tpu_sc as plsc

_ROWS = 4096
_COLS = 200
_NW = 32
_ROWS_W = _ROWS // _NW
_CHUNK = 64
_LANES = 16
_COL_OFFS = (0, 16, 32, 48, 64, 80, 96, 112, 128, 144, 160, 176, 184)


def _compiler_params():
    cp = pltpu.CompilerParams()
    if "needs_layout_passes" in pltpu.CompilerParams.__dataclass_fields__:
        cp = dataclasses.replace(cp, needs_layout_passes=False)
    return cp


@jax.jit
def _emb_gather(z, table):
    mesh = plsc.VectorSubcoreMesh(core_axis_name="c", subcore_axis_name="s")

    @functools.partial(
        pl.kernel,
        out_type=jax.ShapeDtypeStruct((_ROWS, _COLS), jnp.float32),
        mesh=mesh,
        scratch_types=[
            pltpu.VMEM((128,), jnp.float32),
            pltpu.VMEM((_CHUNK, _COLS), jnp.int32),
            pltpu.VMEM((_CHUNK, _COLS), jnp.int32),
            pltpu.VMEM((_CHUNK, _COLS), jnp.float32),
            pltpu.VMEM((_CHUNK, _COLS), jnp.float32),
            pltpu.SemaphoreType.DMA,
            pltpu.SemaphoreType.DMA,
            pltpu.SemaphoreType.DMA,
            pltpu.SemaphoreType.DMA,
        ],
        compiler_params=_compiler_params(),
    )
    def body(table_hbm, z_hbm, out_hbm, table_v,
             idx_a, idx_b, val_a, val_b, s_ia, s_ib, s_oa, s_ob):
        wid = lax.axis_index("s") * 2 + lax.axis_index("c")
        r0 = wid * _ROWS_W

        in0 = pltpu.async_copy(z_hbm.at[pl.ds(r0, _CHUNK)], idx_a, s_ia)
        in1 = pltpu.async_copy(z_hbm.at[pl.ds(r0 + _CHUNK, _CHUNK)], idx_b, s_ib)
        pltpu.sync_copy(table_hbm, table_v.at[pl.ds(0, 120)])

        def gather_chunk(idx_v, val_v):
            @plsc.parallel_loop(0, _CHUNK, step=1)
            def _(r):
                @plsc.parallel_loop(0, 192, step=_LANES, unroll=4)
                def _(c):
                    idx = idx_v[r, pl.ds(c, _LANES)]
                    val_v[r, pl.ds(c, _LANES)] = plsc.load_gather(table_v, [idx])

                idx = idx_v[r, pl.ds(184, _LANES)]
                val_v[r, pl.ds(184, _LANES)] = plsc.load_gather(table_v, [idx])

        in0.wait()
        gather_chunk(idx_a, val_a)
        out0 = pltpu.async_copy(val_a, out_hbm.at[pl.ds(r0, _CHUNK)], s_oa)

        in1.wait()
        gather_chunk(idx_b, val_b)
        out1 = pltpu.async_copy(val_b, out_hbm.at[pl.ds(r0 + _CHUNK, _CHUNK)], s_ob)

        out0.wait()
        out1.wait()

    return body(table, z)


def kernel(z, atomic_temp):
    return _emb_gather(z.astype(jnp.int32), atomic_temp)


# speedup vs baseline: 1.0571x; 1.0571x over previous
import dataclasses
import functools

import jax
import jax.numpy as jnp
from jax import lax
from jax.experimental import pallas as pl
from jax.experimental.pallas import tpu as pltpu
from jax.experimental.pallas import tpu_sc as plsc

_ROWS = 4096
_COLS = 200
_NW = 32
_ROWS_W = _ROWS // _NW
_CHUNK = 64
_LANES = 16
_COL_OFFS = (0, 16, 32, 48, 64, 80, 96, 112, 128, 144, 160, 176, 184)


def _compiler_params():
    cp = pltpu.CompilerParams()
    if "needs_layout_passes" in pltpu.CompilerParams.__dataclass_fields__:
        cp = dataclasses.replace(cp, needs_layout_passes=False)
    return cp


@jax.jit
def _emb_gather(z, table):
    mesh = plsc.VectorSubcoreMesh(core_axis_name="c", subcore_axis_name="s")

    @functools.partial(
        pl.kernel,
        out_type=jax.ShapeDtypeStruct((_ROWS, _COLS), jnp.float32),
        mesh=mesh,
        scratch_types=[
            pltpu.VMEM((128,), jnp.float32),
            pltpu.VMEM((_CHUNK, _COLS), jnp.int32),
            pltpu.VMEM((_CHUNK, _COLS), jnp.int32),
            pltpu.VMEM((_CHUNK, _COLS), jnp.float32),
            pltpu.VMEM((_CHUNK, _COLS), jnp.float32),
            pltpu.SemaphoreType.DMA,
            pltpu.SemaphoreType.DMA,
            pltpu.SemaphoreType.DMA,
            pltpu.SemaphoreType.DMA,
        ],
        compiler_params=_compiler_params(),
    )
    def body(table_hbm, z_hbm, out_hbm, table_v,
             idx_a, idx_b, val_a, val_b, s_ia, s_ib, s_oa, s_ob):
        wid = lax.axis_index("s") * 2 + lax.axis_index("c")
        r0 = wid * _ROWS_W

        in0 = pltpu.async_copy(z_hbm.at[pl.ds(r0, _CHUNK)], idx_a, s_ia)
        in1 = pltpu.async_copy(z_hbm.at[pl.ds(r0 + _CHUNK, _CHUNK)], idx_b, s_ib)
        pltpu.sync_copy(table_hbm, table_v.at[pl.ds(0, 120)])

        def gather_chunk(idx_v, val_v):
            @plsc.parallel_loop(0, _CHUNK, step=1, unroll=2)
            def _(r):
                for c in _COL_OFFS:
                    idx = idx_v[r, pl.ds(c, _LANES)]
                    val_v[r, pl.ds(c, _LANES)] = plsc.load_gather(table_v, [idx])

        in0.wait()
        gather_chunk(idx_a, val_a)
        out0 = pltpu.async_copy(val_a, out_hbm.at[pl.ds(r0, _CHUNK)], s_oa)

        in1.wait()
        gather_chunk(idx_b, val_b)
        out1 = pltpu.async_copy(val_b, out_hbm.at[pl.ds(r0 + _CHUNK, _CHUNK)], s_ob)

        out0.wait()
        out1.wait()

    return body(table, z)


def kernel(z, atomic_temp):
    return _emb_gather(z.astype(jnp.int32), atomic_temp)
